# Initial kernel scaffold; baseline (speedup 1.0000x reference)
import jax
import jax.numpy as jnp
from jax.experimental import pallas as pl


def kernel(x, table, anchors_a, anchors_b):
    raise NotImplementedError("write your pallas kernel here")


# speedup vs baseline: 20.3179x; 20.3179x over previous
import jax
import jax.numpy as jnp
from jax.experimental import pallas as pl

_C = 8


def _tc_body(x_ref, s_ref, p_ref, tab_ref, o_ref):
    d = jax.lax.dot_general(
        x_ref[...], s_ref[...],
        dimension_numbers=(((1,), (0,)), ((), ())),
        precision=jax.lax.Precision.HIGHEST,
        preferred_element_type=jnp.float32)
    sgn = jnp.where(d > 0.0, 1.0, -1.0).astype(jnp.bfloat16)
    m = jax.lax.dot_general(
        sgn, p_ref[...],
        dimension_numbers=(((1,), (0,)), ((), ())),
        preferred_element_type=jnp.float32)
    w = (m == float(_C)).astype(jnp.bfloat16)
    o_ref[...] = jax.lax.dot_general(
        w, tab_ref[...],
        dimension_numbers=(((1,), (0,)), ((), ())),
        preferred_element_type=jnp.float32)


def kernel(x, table, anchors_a, anchors_b):
    B, F = x.shape
    T, R, D = table.shape
    C = _C
    af = anchors_a.astype(jnp.int32).reshape(-1)
    bf = anchors_b.astype(jnp.int32).reshape(-1)
    col = jnp.arange(F, dtype=jnp.int32)[:, None]
    S = ((col == af[None, :]).astype(jnp.float32)
         - (col == bf[None, :]).astype(jnp.float32))
    tc = jnp.arange(T * C, dtype=jnp.int32)
    tr = jnp.arange(T * R, dtype=jnp.int32)
    same_t = (tc[:, None] // C) == (tr[None, :] // R)
    rbit = ((tr[None, :] % R) >> (tc[:, None] % C)) & 1
    P = jnp.where(same_t,
                  jnp.where(rbit == 1, 1.0, -1.0),
                  0.0).astype(jnp.bfloat16)
    tab = table.reshape(T * R, D).astype(jnp.bfloat16)

    BB = 512
    out = pl.pallas_call(
        _tc_body,
        grid=(B // BB,),
        in_specs=[
            pl.BlockSpec((BB, F), lambda i: (i, 0)),
            pl.BlockSpec((F, T * C), lambda i: (0, 0)),
            pl.BlockSpec((T * C, T * R), lambda i: (0, 0)),
            pl.BlockSpec((T * R, D), lambda i: (0, 0)),
        ],
        out_specs=pl.BlockSpec((BB, D), lambda i: (i, 0)),
        out_shape=jax.ShapeDtypeStruct((B, D), jnp.float32),
    )(x, S, P, tab)
    return out
